# baseline (speedup 1.0000x reference)
import jax
import jax.numpy as jnp
from jax import lax
from jax.experimental import pallas as pl
from jax.experimental.pallas import tpu as pltpu
from jax.experimental.pallas import tpu_sc as plsc

N_NODES = 50000
NW = 32
RPW = 1568
NP = NW * RPW
G = 128
NEG = -3.0e38
BLK = 512

_HI = lax.Precision.HIGHEST


def _dot(a, b):
    return jnp.dot(a, b, preferred_element_type=jnp.float32, precision=_HI)


def _uv_body(a_ref, wd_ref, wb_ref, u_ref, v_ref):
    a = a_ref[...]
    u_ref[...] = _dot(a, wd_ref[...])
    v_ref[...] = _dot(a, wb_ref[...])


def _tc_uv(a, wd, wb):
    n, k = a.shape
    return pl.pallas_call(
        _uv_body,
        grid=(n // BLK,),
        in_specs=[pl.BlockSpec((BLK, k), lambda i: (i, 0)),
                  pl.BlockSpec((k, 64), lambda i: (0, 0)),
                  pl.BlockSpec((k, 64), lambda i: (0, 0))],
        out_specs=[pl.BlockSpec((BLK, 64), lambda i: (i, 0)),
                   pl.BlockSpec((BLK, 64), lambda i: (i, 0))],
        out_shape=[jax.ShapeDtypeStruct((n, 64), jnp.float32)] * 2,
    )(a, wd, wb)


def _mid_body(u_ref, m_ref, b_ref, wd_ref, wb_ref, uo_ref, vo_ref):
    a = jnp.maximum(u_ref[...] + b_ref[0:1, :] + m_ref[...], 0.0)
    uo_ref[...] = _dot(a, wd_ref[...])
    vo_ref[...] = _dot(a, wb_ref[...])


def _tc_mid(u, m, b2d, wd, wb):
    return pl.pallas_call(
        _mid_body,
        grid=(NP // BLK,),
        in_specs=[pl.BlockSpec((BLK, 64), lambda i: (i, 0)),
                  pl.BlockSpec((BLK, 64), lambda i: (i, 0)),
                  pl.BlockSpec((8, 64), lambda i: (0, 0)),
                  pl.BlockSpec((64, 64), lambda i: (0, 0)),
                  pl.BlockSpec((64, 64), lambda i: (0, 0))],
        out_specs=[pl.BlockSpec((BLK, 64), lambda i: (i, 0)),
                   pl.BlockSpec((BLK, 64), lambda i: (i, 0))],
        out_shape=[jax.ShapeDtypeStruct((NP, 64), jnp.float32)] * 2,
    )(u, m, b2d, wd, wb)


def _x5g_body(u_ref, m_ref, b_ref, x5_ref, g_ref):
    i = pl.program_id(0)
    x5 = jnp.maximum(u_ref[...] + b_ref[0:1, :] + m_ref[...], 0.0)
    x5_ref[...] = x5
    pm = jnp.broadcast_to(jnp.max(x5, axis=0, keepdims=True), (8, 64))

    @pl.when(i == 0)
    def _():
        g_ref[...] = pm

    @pl.when(i > 0)
    def _():
        g_ref[...] = jnp.maximum(g_ref[...], pm)


def _tc_x5g(u, m, b2d):
    return pl.pallas_call(
        _x5g_body,
        grid=(NP // BLK,),
        in_specs=[pl.BlockSpec((BLK, 64), lambda i: (i, 0)),
                  pl.BlockSpec((BLK, 64), lambda i: (i, 0)),
                  pl.BlockSpec((8, 64), lambda i: (0, 0))],
        out_specs=[pl.BlockSpec((BLK, 64), lambda i: (i, 0)),
                   pl.BlockSpec((8, 64), lambda i: (0, 0))],
        out_shape=[jax.ShapeDtypeStruct((NP, 64), jnp.float32),
                   jax.ShapeDtypeStruct((8, 64), jnp.float32)],
    )(u, m, b2d)


def _fin_body(x5_ref, g_ref, xp_ref, a_ref, bm_ref, bl1_ref, wl2_ref,
              bl2_ref, out_ref):
    gb = _dot(g_ref[0:1, :], bm_ref[...])
    h = jnp.maximum(_dot(x5_ref[...], a_ref[...]) + gb + bl1_ref[0:1, :], 0.0)
    out_ref[...] = xp_ref[...] + _dot(h, wl2_ref[...]) + bl2_ref[0:1, :]


def _tc_final(x5, g8, xpad, amat, bmat, bl1_2d, wl2p, bl2_2d):
    return pl.pallas_call(
        _fin_body,
        grid=(NP // BLK,),
        in_specs=[pl.BlockSpec((BLK, 64), lambda i: (i, 0)),
                  pl.BlockSpec((8, 64), lambda i: (0, 0)),
                  pl.BlockSpec((BLK, 8), lambda i: (i, 0)),
                  pl.BlockSpec((64, 128), lambda i: (0, 0)),
                  pl.BlockSpec((64, 128), lambda i: (0, 0)),
                  pl.BlockSpec((8, 128), lambda i: (0, 0)),
                  pl.BlockSpec((128, 8), lambda i: (0, 0)),
                  pl.BlockSpec((8, 8), lambda i: (0, 0))],
        out_specs=pl.BlockSpec((BLK, 8), lambda i: (i, 0)),
        out_shape=jax.ShapeDtypeStruct((NP, 8), jnp.float32),
    )(x5, g8, xpad, amat, bmat, bl1_2d, wl2p, bl2_2d)


E_EDGES = 800000
SG = 3200
FG = 4096
FB = FG + 288


def _filter_body(src_hbm, dst_hbm, srcl_hbm, dlocl_hbm, cnt_hbm,
                 sbuf, dbuf, stgs, stgd, cntv, sem_a, sem_b):
    wid = lax.axis_index("s") * 2 + lax.axis_index("c")
    lo = wid * RPW
    hi = lo + RPW
    nsc = E_EDGES // SG
    sems = (sem_a, sem_b)

    def in_dma(t, b):
        return (pltpu.make_async_copy(src_hbm.at[pl.ds(t * SG, SG)],
                                      sbuf.at[b], sems[b]),
                pltpu.make_async_copy(dst_hbm.at[pl.ds(t * SG, SG)],
                                      dbuf.at[b], sems[b]))

    for t0, b0 in ((0, 0), (1, 1)):
        sa, da = in_dma(t0, b0)
        sa.start()
        da.start()

    def do_flush(c):
        ptr, off = c
        offa = pl.multiple_of(off, FG)
        pltpu.sync_copy(stgs.at[pl.ds(0, FG)],
                        srcl_hbm.at[wid, pl.ds(offa, FG)])
        pltpu.sync_copy(stgd.at[pl.ds(0, FG)],
                        dlocl_hbm.at[wid, pl.ds(offa, FG)])
        for k in range(9):
            ts = stgs[pl.ds(FG + k * 16, 16)]
            td = stgd[pl.ds(FG + k * 16, 16)]
            stgs[pl.ds(k * 16, 16)] = ts
            stgd[pl.ds(k * 16, 16)] = td
        return ptr - FG, off + FG

    def super_body(t, carry):
        for b in range(2):

            def blk(g8, c):
                ptr, off = c
                for j in range(8):
                    g16 = pl.multiple_of(g8 * 128 + j * 16, 16)
                    s16 = sbuf[b, pl.ds(g16, 16)]
                    d16 = dbuf[b, pl.ds(g16, 16)]
                    dl16 = d16 - lo
                    msk = dl16.astype(jnp.uint32) < jnp.uint32(RPW)
                    plsc.store_compressed(stgs.at[pl.ds(ptr, 16)], s16,
                                          mask=msk)
                    plsc.store_compressed(stgd.at[pl.ds(ptr, 16)], dl16,
                                          mask=msk)
                    pc = plsc.all_reduce_population_count(msk)
                    ptr = ptr + pc[0]
                return lax.cond(ptr >= FG, do_flush, lambda c: c, (ptr, off))

            ts = 2 * t + b
            sa, da = in_dma(ts, b)
            sa.wait()
            da.wait()
            carry = lax.fori_loop(0, SG // 128, blk, carry)

            @pl.when(ts + 2 < nsc)
            def _():
                sn, dn = in_dma(ts + 2, b)
                sn.start()
                dn.start()

        return carry

    ptr, off = lax.fori_loop(0, nsc // 2, super_body, (0, 0))

    zs = jnp.zeros((16,), jnp.int32)
    zd = jnp.full((16,), RPW, jnp.int32)
    for k in range(9):
        stgs[pl.ds(ptr + k * 16, 16)] = zs
        stgd[pl.ds(ptr + k * 16, 16)] = zd
    cntp = lax.div(ptr + 127, 128) * 128

    def final_flush(k, c):
        fo = pl.multiple_of(off + k * 128, 128)
        ko = pl.multiple_of(k * 128, 128)
        pltpu.sync_copy(stgs.at[pl.ds(ko, 128)],
                        srcl_hbm.at[wid, pl.ds(fo, 128)])
        pltpu.sync_copy(stgd.at[pl.ds(ko, 128)],
                        dlocl_hbm.at[wid, pl.ds(fo, 128)])
        return c

    lax.fori_loop(0, cntp // 128, final_flush, 0)
    cntv[...] = lax.broadcast(off + cntp, (16,))
    pltpu.sync_copy(cntv, cnt_hbm.at[wid])


def _sc_filter(src, dst):
    mesh = plsc.VectorSubcoreMesh(core_axis_name="c", subcore_axis_name="s")
    kfn = pl.kernel(
        _filter_body,
        out_type=[jax.ShapeDtypeStruct((NW, E_EDGES), jnp.int32),
                  jax.ShapeDtypeStruct((NW, E_EDGES), jnp.int32),
                  jax.ShapeDtypeStruct((NW, 16), jnp.int32)],
        mesh=mesh,
        scratch_types=[
            pltpu.VMEM((2, SG), jnp.int32),
            pltpu.VMEM((2, SG), jnp.int32),
            pltpu.VMEM((FB,), jnp.int32),
            pltpu.VMEM((FB,), jnp.int32),
            pltpu.VMEM((16,), jnp.int32),
            pltpu.SemaphoreType.DMA,
            pltpu.SemaphoreType.DMA,
        ],
        compiler_params=pltpu.CompilerParams(use_tc_tiling_on_sc=False,
                                             needs_layout_passes=False),
    )
    return kfn(src, dst)


def _segmax_body(v_hbm, srcl_hbm, dlocl_hbm, cnt_hbm, m_hbm,
                 idx_v, rows_v, dstv_v, acc0, acc1, acc2, acc3, stv_v,
                 sem_g0, sem_g1, sem_i0, sem_i1, sem_d0, sem_d1):
    wid = lax.axis_index("s") * 2 + lax.axis_index("c")
    lo = wid * RPW
    pltpu.sync_copy(cnt_hbm.at[wid], stv_v)

    accs = (acc0, acc1, acc2, acc3)
    neg = jnp.full((16,), NEG, jnp.float32)

    @pl.loop(0, RPW + 16)
    def _(r):
        for f in range(4):
            accs[f][r, :] = neg

    cnt = stv_v[pl.ds(0, 16)]
    c0 = 0
    c1 = cnt[0] // G

    sem_g = (sem_g0, sem_g1)
    sem_i = (sem_i0, sem_i1)
    sem_d = (sem_d0, sem_d1)

    def idx_dma(ci, b):
        co = pl.multiple_of(ci * G, G)
        return (pltpu.make_async_copy(srcl_hbm.at[wid, pl.ds(co, G)],
                                      idx_v.at[b], sem_i[b]),
                pltpu.make_async_copy(dlocl_hbm.at[wid, pl.ds(co, G)],
                                      dstv_v.at[b], sem_d[b]))

    def gather(b):
        return pltpu.make_async_copy(v_hbm.at[idx_v.at[b]], rows_v.at[b],
                                     sem_g[b])

    def process(ci, b):
        def grp(gi, carry2):
            d16 = dstv_v[b, pl.ds(gi * 16, 16)]
            for lane in range(16):
                dl = d16[lane]
                ei = gi * 16 + lane
                for f in range(4):
                    sl = pl.ds(f * 16, 16)
                    accs[f][dl, :] = jnp.maximum(accs[f][dl, :],
                                                 rows_v[b, ei, sl])

            return carry2

        lax.fori_loop(0, G // 16, grp, 0)

    @pl.when(c0 < c1)
    def _():
        ia, da = idx_dma(c0, 0)
        ia.start()
        da.start()
        ia.wait()
        gather(0).start()

        @pl.when(c0 + 1 < c1)
        def _():
            ib, db = idx_dma(c0 + 1, 1)
            ib.start()
            db.start()

        def pair(t, carry):
            for b in range(2):
                ci = c0 + 2 * t + b

                @pl.when(ci < c1)
                def _():
                    gather(b).wait()
                    _, dw = idx_dma(ci, b)
                    dw.wait()

                    @pl.when(ci + 1 < c1)
                    def _():
                        iw, _ = idx_dma(ci + 1, 1 - b)
                        iw.wait()
                        gather(1 - b).start()

                    @pl.when(ci + 2 < c1)
                    def _():
                        inx, _ = idx_dma(ci + 2, b)
                        inx.start()

                    process(ci, b)

                    @pl.when(ci + 2 < c1)
                    def _():
                        _, dnx = idx_dma(ci + 2, b)
                        dnx.start()

            return carry

        npairs = lax.div(c1 - c0 + 1, 2)
        lax.fori_loop(0, npairs, pair, 0)

    for f in range(4):
        pltpu.sync_copy(accs[f].at[pl.ds(0, RPW)],
                        m_hbm.at[pl.ds(lo, RPW), pl.ds(f * 16, 16)])


def _sc_segmax(v, srcl, dlocl, cnt):
    mesh = plsc.VectorSubcoreMesh(core_axis_name="c", subcore_axis_name="s")
    kfn = pl.kernel(
        _segmax_body,
        out_type=jax.ShapeDtypeStruct((NP, 64), jnp.float32),
        mesh=mesh,
        scratch_types=[
            pltpu.VMEM((2, G), jnp.int32),
            pltpu.VMEM((2, G, 64), jnp.float32),
            pltpu.VMEM((2, G), jnp.int32),
            pltpu.VMEM((RPW + 16, 16), jnp.float32),
            pltpu.VMEM((RPW + 16, 16), jnp.float32),
            pltpu.VMEM((RPW + 16, 16), jnp.float32),
            pltpu.VMEM((RPW + 16, 16), jnp.float32),
            pltpu.VMEM((16,), jnp.int32),
            pltpu.SemaphoreType.DMA,
            pltpu.SemaphoreType.DMA,
            pltpu.SemaphoreType.DMA,
            pltpu.SemaphoreType.DMA,
            pltpu.SemaphoreType.DMA,
            pltpu.SemaphoreType.DMA,
        ],
        compiler_params=pltpu.CompilerParams(use_tc_tiling_on_sc=False),
    )
    return kfn(v, srcl, dlocl, cnt)


def kernel(x, edge_index, W1, b1, W2, b2, W3, b3, W4, b4, W5, b5,
           Wl1, bl1, Wl2, bl2):
    f32 = jnp.float32
    src = edge_index[0].astype(jnp.int32)
    dst = edge_index[1].astype(jnp.int32)

    srcl, dlocl, cnt = _sc_filter(src, dst)

    xpad = jnp.zeros((NP, 8), f32).at[:N_NODES, :3].set(x)

    wd1 = jnp.zeros((8, 64), f32).at[:3].set(W1[:3] - W1[3:])
    wb1 = jnp.zeros((8, 64), f32).at[:3].set(W1[3:])
    u, v = _tc_uv(xpad, wd1, wb1)
    m = _sc_segmax(v, srcl, dlocl, cnt)
    bprev = b1

    for (W, b) in ((W2, b2), (W3, b3), (W4, b4), (W5, b5)):
        wd = W[:64] - W[64:]
        wb = W[64:]
        b2d = jnp.broadcast_to(bprev.reshape(1, 64), (8, 64))
        u, v = _tc_mid(u, m, b2d, wd, wb)
        m = _sc_segmax(v, srcl, dlocl, cnt)
        bprev = b

    b2d5 = jnp.broadcast_to(bprev.reshape(1, 64), (8, 64))
    x5, g8 = _tc_x5g(u, m, b2d5)

    amat = Wl1[:64]
    bmat = Wl1[64:]
    bl1_2d = jnp.broadcast_to(bl1.reshape(1, 128), (8, 128))
    wl2p = jnp.zeros((128, 8), f32).at[:, :3].set(Wl2)
    bl2_2d = jnp.zeros((8, 8), f32).at[:, :3].set(
        jnp.broadcast_to(bl2.reshape(1, 3), (8, 3)))
    outp = _tc_final(x5, g8, xpad, amat, bmat, bl1_2d, wl2p, bl2_2d)
    return outp[:N_NODES, :3]


# speedup vs baseline: 1.0018x; 1.0018x over previous
import jax
import jax.numpy as jnp
from jax import lax
from jax.experimental import pallas as pl
from jax.experimental.pallas import tpu as pltpu
from jax.experimental.pallas import tpu_sc as plsc

N_NODES = 50000
NW = 32
RPW = 1568
NP = NW * RPW
G = 128
NEG = -3.0e38
BLK = 512

_HI = lax.Precision.HIGHEST


def _dot(a, b):
    return jnp.dot(a, b, preferred_element_type=jnp.float32, precision=_HI)


def _uv_body(a_ref, wd_ref, wb_ref, u_ref, v_ref):
    a = a_ref[...]
    u_ref[...] = _dot(a, wd_ref[...])
    v_ref[...] = _dot(a, wb_ref[...])


def _tc_uv(a, wd, wb):
    n, k = a.shape
    return pl.pallas_call(
        _uv_body,
        grid=(n // BLK,),
        in_specs=[pl.BlockSpec((BLK, k), lambda i: (i, 0)),
                  pl.BlockSpec((k, 64), lambda i: (0, 0)),
                  pl.BlockSpec((k, 64), lambda i: (0, 0))],
        out_specs=[pl.BlockSpec((BLK, 64), lambda i: (i, 0)),
                   pl.BlockSpec((BLK, 64), lambda i: (i, 0))],
        out_shape=[jax.ShapeDtypeStruct((n, 64), jnp.float32)] * 2,
    )(a, wd, wb)


def _mid_body(u_ref, m_ref, b_ref, wd_ref, wb_ref, uo_ref, vo_ref):
    a = jnp.maximum(u_ref[...] + b_ref[0:1, :] + m_ref[...], 0.0)
    uo_ref[...] = _dot(a, wd_ref[...])
    vo_ref[...] = _dot(a, wb_ref[...])


def _tc_mid(u, m, b2d, wd, wb):
    return pl.pallas_call(
        _mid_body,
        grid=(NP // BLK,),
        in_specs=[pl.BlockSpec((BLK, 64), lambda i: (i, 0)),
                  pl.BlockSpec((BLK, 64), lambda i: (i, 0)),
                  pl.BlockSpec((8, 64), lambda i: (0, 0)),
                  pl.BlockSpec((64, 64), lambda i: (0, 0)),
                  pl.BlockSpec((64, 64), lambda i: (0, 0))],
        out_specs=[pl.BlockSpec((BLK, 64), lambda i: (i, 0)),
                   pl.BlockSpec((BLK, 64), lambda i: (i, 0))],
        out_shape=[jax.ShapeDtypeStruct((NP, 64), jnp.float32)] * 2,
    )(u, m, b2d, wd, wb)


def _x5g_body(u_ref, m_ref, b_ref, x5_ref, g_ref):
    i = pl.program_id(0)
    x5 = jnp.maximum(u_ref[...] + b_ref[0:1, :] + m_ref[...], 0.0)
    x5_ref[...] = x5
    pm = jnp.broadcast_to(jnp.max(x5, axis=0, keepdims=True), (8, 64))

    @pl.when(i == 0)
    def _():
        g_ref[...] = pm

    @pl.when(i > 0)
    def _():
        g_ref[...] = jnp.maximum(g_ref[...], pm)


def _tc_x5g(u, m, b2d):
    return pl.pallas_call(
        _x5g_body,
        grid=(NP // BLK,),
        in_specs=[pl.BlockSpec((BLK, 64), lambda i: (i, 0)),
                  pl.BlockSpec((BLK, 64), lambda i: (i, 0)),
                  pl.BlockSpec((8, 64), lambda i: (0, 0))],
        out_specs=[pl.BlockSpec((BLK, 64), lambda i: (i, 0)),
                   pl.BlockSpec((8, 64), lambda i: (0, 0))],
        out_shape=[jax.ShapeDtypeStruct((NP, 64), jnp.float32),
                   jax.ShapeDtypeStruct((8, 64), jnp.float32)],
    )(u, m, b2d)


def _fin_body(x5_ref, g_ref, xp_ref, a_ref, bm_ref, bl1_ref, wl2_ref,
              bl2_ref, out_ref):
    gb = _dot(g_ref[0:1, :], bm_ref[...])
    h = jnp.maximum(_dot(x5_ref[...], a_ref[...]) + gb + bl1_ref[0:1, :], 0.0)
    out_ref[...] = xp_ref[...] + _dot(h, wl2_ref[...]) + bl2_ref[0:1, :]


def _tc_final(x5, g8, xpad, amat, bmat, bl1_2d, wl2p, bl2_2d):
    return pl.pallas_call(
        _fin_body,
        grid=(NP // BLK,),
        in_specs=[pl.BlockSpec((BLK, 64), lambda i: (i, 0)),
                  pl.BlockSpec((8, 64), lambda i: (0, 0)),
                  pl.BlockSpec((BLK, 8), lambda i: (i, 0)),
                  pl.BlockSpec((64, 128), lambda i: (0, 0)),
                  pl.BlockSpec((64, 128), lambda i: (0, 0)),
                  pl.BlockSpec((8, 128), lambda i: (0, 0)),
                  pl.BlockSpec((128, 8), lambda i: (0, 0)),
                  pl.BlockSpec((8, 8), lambda i: (0, 0))],
        out_specs=pl.BlockSpec((BLK, 8), lambda i: (i, 0)),
        out_shape=jax.ShapeDtypeStruct((NP, 8), jnp.float32),
    )(x5, g8, xpad, amat, bmat, bl1_2d, wl2p, bl2_2d)


E_EDGES = 800000
SG = 3200
FG = 4096
FB = FG + 288


def _filter_body(src_hbm, dst_hbm, srcl_hbm, dlocl_hbm, cnt_hbm,
                 sbuf, dbuf, stgs, stgd, cntv, sem_a, sem_b):
    wid = lax.axis_index("s") * 2 + lax.axis_index("c")
    lo = wid * RPW
    hi = lo + RPW
    nsc = E_EDGES // SG
    sems = (sem_a, sem_b)

    def in_dma(t, b):
        return (pltpu.make_async_copy(src_hbm.at[pl.ds(t * SG, SG)],
                                      sbuf.at[b], sems[b]),
                pltpu.make_async_copy(dst_hbm.at[pl.ds(t * SG, SG)],
                                      dbuf.at[b], sems[b]))

    for t0, b0 in ((0, 0), (1, 1)):
        sa, da = in_dma(t0, b0)
        sa.start()
        da.start()

    def do_flush(c):
        ptr, off = c
        offa = pl.multiple_of(off, FG)
        pltpu.sync_copy(stgs.at[pl.ds(0, FG)],
                        srcl_hbm.at[wid, pl.ds(offa, FG)])
        pltpu.sync_copy(stgd.at[pl.ds(0, FG)],
                        dlocl_hbm.at[wid, pl.ds(offa, FG)])
        for k in range(9):
            ts = stgs[pl.ds(FG + k * 16, 16)]
            td = stgd[pl.ds(FG + k * 16, 16)]
            stgs[pl.ds(k * 16, 16)] = ts
            stgd[pl.ds(k * 16, 16)] = td
        return ptr - FG, off + FG

    def super_body(t, carry):
        for b in range(2):

            def blk(g8, c):
                ptr, off = c
                for j in range(8):
                    g16 = pl.multiple_of(g8 * 128 + j * 16, 16)
                    s16 = sbuf[b, pl.ds(g16, 16)]
                    d16 = dbuf[b, pl.ds(g16, 16)]
                    dl16 = d16 - lo
                    msk = dl16.astype(jnp.uint32) < jnp.uint32(RPW)
                    plsc.store_compressed(stgs.at[pl.ds(ptr, 16)], s16,
                                          mask=msk)
                    plsc.store_compressed(stgd.at[pl.ds(ptr, 16)], dl16,
                                          mask=msk)
                    pc = plsc.all_reduce_population_count(msk)
                    ptr = ptr + pc[0]
                return lax.cond(ptr >= FG, do_flush, lambda c: c, (ptr, off))

            ts = 2 * t + b
            sa, da = in_dma(ts, b)
            sa.wait()
            da.wait()
            carry = lax.fori_loop(0, SG // 128, blk, carry)

            @pl.when(ts + 2 < nsc)
            def _():
                sn, dn = in_dma(ts + 2, b)
                sn.start()
                dn.start()

        return carry

    ptr, off = lax.fori_loop(0, nsc // 2, super_body, (0, 0))

    zs = jnp.zeros((16,), jnp.int32)
    zd = jnp.full((16,), RPW, jnp.int32)
    for k in range(9):
        stgs[pl.ds(ptr + k * 16, 16)] = zs
        stgd[pl.ds(ptr + k * 16, 16)] = zd
    cntp = lax.div(ptr + 127, 128) * 128

    def final_flush(k, c):
        fo = pl.multiple_of(off + k * 128, 128)
        ko = pl.multiple_of(k * 128, 128)
        pltpu.sync_copy(stgs.at[pl.ds(ko, 128)],
                        srcl_hbm.at[wid, pl.ds(fo, 128)])
        pltpu.sync_copy(stgd.at[pl.ds(ko, 128)],
                        dlocl_hbm.at[wid, pl.ds(fo, 128)])
        return c

    lax.fori_loop(0, cntp // 128, final_flush, 0)
    cntv[...] = lax.broadcast(off + cntp, (16,))
    pltpu.sync_copy(cntv, cnt_hbm.at[wid])


def _sc_filter(src, dst):
    mesh = plsc.VectorSubcoreMesh(core_axis_name="c", subcore_axis_name="s")
    kfn = pl.kernel(
        _filter_body,
        out_type=[jax.ShapeDtypeStruct((NW, E_EDGES), jnp.int32),
                  jax.ShapeDtypeStruct((NW, E_EDGES), jnp.int32),
                  jax.ShapeDtypeStruct((NW, 16), jnp.int32)],
        mesh=mesh,
        scratch_types=[
            pltpu.VMEM((2, SG), jnp.int32),
            pltpu.VMEM((2, SG), jnp.int32),
            pltpu.VMEM((FB,), jnp.int32),
            pltpu.VMEM((FB,), jnp.int32),
            pltpu.VMEM((16,), jnp.int32),
            pltpu.SemaphoreType.DMA,
            pltpu.SemaphoreType.DMA,
        ],
        compiler_params=pltpu.CompilerParams(use_tc_tiling_on_sc=False,
                                             needs_layout_passes=False),
    )
    return kfn(src, dst)


def _segmax_body(v_hbm, srcl_hbm, dlocl_hbm, cnt_hbm, m_hbm,
                 idx_v, rows_v, dstv_v, acc0, acc1, acc2, acc3, stv_v,
                 sem_g0, sem_g1, sem_i0, sem_i1, sem_d0, sem_d1):
    wid = lax.axis_index("s") * 2 + lax.axis_index("c")
    lo = wid * RPW
    pltpu.sync_copy(cnt_hbm.at[wid], stv_v)

    accs = (acc0, acc1, acc2, acc3)
    neg = jnp.full((16,), NEG, jnp.float32)

    @pl.loop(0, RPW + 16)
    def _(r):
        for f in range(4):
            accs[f][r, :] = neg

    cnt = stv_v[pl.ds(0, 16)]
    c0 = 0
    c1 = cnt[0] // G

    sem_g = (sem_g0, sem_g1)
    sem_i = (sem_i0, sem_i1)
    sem_d = (sem_d0, sem_d1)

    def idx_dma(ci, b):
        co = pl.multiple_of(ci * G, G)
        return (pltpu.make_async_copy(srcl_hbm.at[wid, pl.ds(co, G)],
                                      idx_v.at[b], sem_i[b]),
                pltpu.make_async_copy(dlocl_hbm.at[wid, pl.ds(co, G)],
                                      dstv_v.at[b], sem_d[b]))

    def gather(b):
        return pltpu.make_async_copy(v_hbm.at[idx_v.at[b]], rows_v.at[b],
                                     sem_g[b])

    def process(ci, b):
        def grp(gi, carry2):
            d16 = dstv_v[b, pl.ds(gi * 16, 16)]
            for lane in range(16):
                dl = d16[lane]
                ei = gi * 16 + lane
                for f in range(4):
                    sl = pl.ds(f * 16, 16)
                    accs[f][dl, :] = jnp.maximum(accs[f][dl, :],
                                                 rows_v[b, ei, sl])

            return carry2

        lax.fori_loop(0, G // 16, grp, 0, unroll=2)

    @pl.when(c0 < c1)
    def _():
        ia, da = idx_dma(c0, 0)
        ia.start()
        da.start()
        ia.wait()
        gather(0).start()

        @pl.when(c0 + 1 < c1)
        def _():
            ib, db = idx_dma(c0 + 1, 1)
            ib.start()
            db.start()

        def pair(t, carry):
            for b in range(2):
                ci = c0 + 2 * t + b

                @pl.when(ci < c1)
                def _():
                    gather(b).wait()
                    _, dw = idx_dma(ci, b)
                    dw.wait()

                    @pl.when(ci + 1 < c1)
                    def _():
                        iw, _ = idx_dma(ci + 1, 1 - b)
                        iw.wait()
                        gather(1 - b).start()

                    @pl.when(ci + 2 < c1)
                    def _():
                        inx, _ = idx_dma(ci + 2, b)
                        inx.start()

                    process(ci, b)

                    @pl.when(ci + 2 < c1)
                    def _():
                        _, dnx = idx_dma(ci + 2, b)
                        dnx.start()

            return carry

        npairs = lax.div(c1 - c0 + 1, 2)
        lax.fori_loop(0, npairs, pair, 0)

    for f in range(4):
        pltpu.sync_copy(accs[f].at[pl.ds(0, RPW)],
                        m_hbm.at[pl.ds(lo, RPW), pl.ds(f * 16, 16)])


def _sc_segmax(v, srcl, dlocl, cnt):
    mesh = plsc.VectorSubcoreMesh(core_axis_name="c", subcore_axis_name="s")
    kfn = pl.kernel(
        _segmax_body,
        out_type=jax.ShapeDtypeStruct((NP, 64), jnp.float32),
        mesh=mesh,
        scratch_types=[
            pltpu.VMEM((2, G), jnp.int32),
            pltpu.VMEM((2, G, 64), jnp.float32),
            pltpu.VMEM((2, G), jnp.int32),
            pltpu.VMEM((RPW + 16, 16), jnp.float32),
            pltpu.VMEM((RPW + 16, 16), jnp.float32),
            pltpu.VMEM((RPW + 16, 16), jnp.float32),
            pltpu.VMEM((RPW + 16, 16), jnp.float32),
            pltpu.VMEM((16,), jnp.int32),
            pltpu.SemaphoreType.DMA,
            pltpu.SemaphoreType.DMA,
            pltpu.SemaphoreType.DMA,
            pltpu.SemaphoreType.DMA,
            pltpu.SemaphoreType.DMA,
            pltpu.SemaphoreType.DMA,
        ],
        compiler_params=pltpu.CompilerParams(use_tc_tiling_on_sc=False),
    )
    return kfn(v, srcl, dlocl, cnt)


def kernel(x, edge_index, W1, b1, W2, b2, W3, b3, W4, b4, W5, b5,
           Wl1, bl1, Wl2, bl2):
    f32 = jnp.float32
    src = edge_index[0].astype(jnp.int32)
    dst = edge_index[1].astype(jnp.int32)

    srcl, dlocl, cnt = _sc_filter(src, dst)

    xpad = jnp.zeros((NP, 8), f32).at[:N_NODES, :3].set(x)

    wd1 = jnp.zeros((8, 64), f32).at[:3].set(W1[:3] - W1[3:])
    wb1 = jnp.zeros((8, 64), f32).at[:3].set(W1[3:])
    u, v = _tc_uv(xpad, wd1, wb1)
    m = _sc_segmax(v, srcl, dlocl, cnt)
    bprev = b1

    for (W, b) in ((W2, b2), (W3, b3), (W4, b4), (W5, b5)):
        wd = W[:64] - W[64:]
        wb = W[64:]
        b2d = jnp.broadcast_to(bprev.reshape(1, 64), (8, 64))
        u, v = _tc_mid(u, m, b2d, wd, wb)
        m = _sc_segmax(v, srcl, dlocl, cnt)
        bprev = b

    b2d5 = jnp.broadcast_to(bprev.reshape(1, 64), (8, 64))
    x5, g8 = _tc_x5g(u, m, b2d5)

    amat = Wl1[:64]
    bmat = Wl1[64:]
    bl1_2d = jnp.broadcast_to(bl1.reshape(1, 128), (8, 128))
    wl2p = jnp.zeros((128, 8), f32).at[:, :3].set(Wl2)
    bl2_2d = jnp.zeros((8, 8), f32).at[:, :3].set(
        jnp.broadcast_to(bl2.reshape(1, 3), (8, 3)))
    outp = _tc_final(x5, g8, xpad, amat, bmat, bl1_2d, wl2p, bl2_2d)
    return outp[:N_NODES, :3]


# speedup vs baseline: 1.0089x; 1.0071x over previous
import jax
import jax.numpy as jnp
from jax import lax
from jax.experimental import pallas as pl
from jax.experimental.pallas import tpu as pltpu
from jax.experimental.pallas import tpu_sc as plsc

N_NODES = 50000
NW = 32
RPW = 1568
NP = NW * RPW
G = 128
NEG = -3.0e38
BLK = 512

_HI = lax.Precision.HIGHEST


def _dot(a, b):
    return jnp.dot(a, b, preferred_element_type=jnp.float32, precision=_HI)


def _uv_body(a_ref, wd_ref, wb_ref, u_ref, v_ref):
    a = a_ref[...]
    u_ref[...] = _dot(a, wd_ref[...])
    v_ref[...] = _dot(a, wb_ref[...])


def _tc_uv(a, wd, wb):
    n, k = a.shape
    return pl.pallas_call(
        _uv_body,
        grid=(n // BLK,),
        in_specs=[pl.BlockSpec((BLK, k), lambda i: (i, 0)),
                  pl.BlockSpec((k, 64), lambda i: (0, 0)),
                  pl.BlockSpec((k, 64), lambda i: (0, 0))],
        out_specs=[pl.BlockSpec((BLK, 64), lambda i: (i, 0)),
                   pl.BlockSpec((BLK, 64), lambda i: (i, 0))],
        out_shape=[jax.ShapeDtypeStruct((n, 64), jnp.float32)] * 2,
    )(a, wd, wb)


def _mid_body(u_ref, m_ref, b_ref, wd_ref, wb_ref, uo_ref, vo_ref):
    a = jnp.maximum(u_ref[...] + b_ref[0:1, :] + m_ref[...], 0.0)
    uo_ref[...] = _dot(a, wd_ref[...])
    vo_ref[...] = _dot(a, wb_ref[...])


def _tc_mid(u, m, b2d, wd, wb):
    return pl.pallas_call(
        _mid_body,
        grid=(NP // BLK,),
        in_specs=[pl.BlockSpec((BLK, 64), lambda i: (i, 0)),
                  pl.BlockSpec((BLK, 64), lambda i: (i, 0)),
                  pl.BlockSpec((8, 64), lambda i: (0, 0)),
                  pl.BlockSpec((64, 64), lambda i: (0, 0)),
                  pl.BlockSpec((64, 64), lambda i: (0, 0))],
        out_specs=[pl.BlockSpec((BLK, 64), lambda i: (i, 0)),
                   pl.BlockSpec((BLK, 64), lambda i: (i, 0))],
        out_shape=[jax.ShapeDtypeStruct((NP, 64), jnp.float32)] * 2,
    )(u, m, b2d, wd, wb)


def _x5g_body(u_ref, m_ref, b_ref, x5_ref, g_ref):
    i = pl.program_id(0)
    x5 = jnp.maximum(u_ref[...] + b_ref[0:1, :] + m_ref[...], 0.0)
    x5_ref[...] = x5
    pm = jnp.broadcast_to(jnp.max(x5, axis=0, keepdims=True), (8, 64))

    @pl.when(i == 0)
    def _():
        g_ref[...] = pm

    @pl.when(i > 0)
    def _():
        g_ref[...] = jnp.maximum(g_ref[...], pm)


def _tc_x5g(u, m, b2d):
    return pl.pallas_call(
        _x5g_body,
        grid=(NP // BLK,),
        in_specs=[pl.BlockSpec((BLK, 64), lambda i: (i, 0)),
                  pl.BlockSpec((BLK, 64), lambda i: (i, 0)),
                  pl.BlockSpec((8, 64), lambda i: (0, 0))],
        out_specs=[pl.BlockSpec((BLK, 64), lambda i: (i, 0)),
                   pl.BlockSpec((8, 64), lambda i: (0, 0))],
        out_shape=[jax.ShapeDtypeStruct((NP, 64), jnp.float32),
                   jax.ShapeDtypeStruct((8, 64), jnp.float32)],
    )(u, m, b2d)


def _fin_body(x5_ref, g_ref, xp_ref, a_ref, bm_ref, bl1_ref, wl2_ref,
              bl2_ref, out_ref):
    gb = _dot(g_ref[0:1, :], bm_ref[...])
    h = jnp.maximum(_dot(x5_ref[...], a_ref[...]) + gb + bl1_ref[0:1, :], 0.0)
    out_ref[...] = xp_ref[...] + _dot(h, wl2_ref[...]) + bl2_ref[0:1, :]


def _tc_final(x5, g8, xpad, amat, bmat, bl1_2d, wl2p, bl2_2d):
    return pl.pallas_call(
        _fin_body,
        grid=(NP // BLK,),
        in_specs=[pl.BlockSpec((BLK, 64), lambda i: (i, 0)),
                  pl.BlockSpec((8, 64), lambda i: (0, 0)),
                  pl.BlockSpec((BLK, 8), lambda i: (i, 0)),
                  pl.BlockSpec((64, 128), lambda i: (0, 0)),
                  pl.BlockSpec((64, 128), lambda i: (0, 0)),
                  pl.BlockSpec((8, 128), lambda i: (0, 0)),
                  pl.BlockSpec((128, 8), lambda i: (0, 0)),
                  pl.BlockSpec((8, 8), lambda i: (0, 0))],
        out_specs=pl.BlockSpec((BLK, 8), lambda i: (i, 0)),
        out_shape=jax.ShapeDtypeStruct((NP, 8), jnp.float32),
    )(x5, g8, xpad, amat, bmat, bl1_2d, wl2p, bl2_2d)


E_EDGES = 800000
SG = 3200
FG = 4096
FB = FG + 288


def _filter_body(src_hbm, dst_hbm, srcl_hbm, dlocl_hbm, cnt_hbm,
                 sbuf, dbuf, stgs, stgd, cntv, sem_a, sem_b):
    wid = lax.axis_index("s") * 2 + lax.axis_index("c")
    lo = wid * RPW
    hi = lo + RPW
    nsc = E_EDGES // SG
    sems = (sem_a, sem_b)

    def in_dma(t, b):
        return (pltpu.make_async_copy(src_hbm.at[pl.ds(t * SG, SG)],
                                      sbuf.at[b], sems[b]),
                pltpu.make_async_copy(dst_hbm.at[pl.ds(t * SG, SG)],
                                      dbuf.at[b], sems[b]))

    for t0, b0 in ((0, 0), (1, 1)):
        sa, da = in_dma(t0, b0)
        sa.start()
        da.start()

    def do_flush(c):
        ptr, off = c
        offa = pl.multiple_of(off, FG)
        pltpu.sync_copy(stgs.at[pl.ds(0, FG)],
                        srcl_hbm.at[wid, pl.ds(offa, FG)])
        pltpu.sync_copy(stgd.at[pl.ds(0, FG)],
                        dlocl_hbm.at[wid, pl.ds(offa, FG)])
        for k in range(9):
            ts = stgs[pl.ds(FG + k * 16, 16)]
            td = stgd[pl.ds(FG + k * 16, 16)]
            stgs[pl.ds(k * 16, 16)] = ts
            stgd[pl.ds(k * 16, 16)] = td
        return ptr - FG, off + FG

    def super_body(t, carry):
        for b in range(2):

            def blk(g8, c):
                ptr, off = c
                for j in range(8):
                    g16 = pl.multiple_of(g8 * 128 + j * 16, 16)
                    s16 = sbuf[b, pl.ds(g16, 16)]
                    d16 = dbuf[b, pl.ds(g16, 16)]
                    dl16 = d16 - lo
                    msk = dl16.astype(jnp.uint32) < jnp.uint32(RPW)
                    plsc.store_compressed(stgs.at[pl.ds(ptr, 16)], s16,
                                          mask=msk)
                    plsc.store_compressed(stgd.at[pl.ds(ptr, 16)], dl16,
                                          mask=msk)
                    pc = plsc.all_reduce_population_count(msk)
                    ptr = ptr + pc[0]
                return lax.cond(ptr >= FG, do_flush, lambda c: c, (ptr, off))

            ts = 2 * t + b
            sa, da = in_dma(ts, b)
            sa.wait()
            da.wait()
            carry = lax.fori_loop(0, SG // 128, blk, carry)

            @pl.when(ts + 2 < nsc)
            def _():
                sn, dn = in_dma(ts + 2, b)
                sn.start()
                dn.start()

        return carry

    ptr, off = lax.fori_loop(0, nsc // 2, super_body, (0, 0))

    zs = jnp.zeros((16,), jnp.int32)
    zd = jnp.full((16,), RPW, jnp.int32)
    for k in range(9):
        stgs[pl.ds(ptr + k * 16, 16)] = zs
        stgd[pl.ds(ptr + k * 16, 16)] = zd
    cntp = lax.div(ptr + 127, 128) * 128

    def final_flush(k, c):
        fo = pl.multiple_of(off + k * 128, 128)
        ko = pl.multiple_of(k * 128, 128)
        pltpu.sync_copy(stgs.at[pl.ds(ko, 128)],
                        srcl_hbm.at[wid, pl.ds(fo, 128)])
        pltpu.sync_copy(stgd.at[pl.ds(ko, 128)],
                        dlocl_hbm.at[wid, pl.ds(fo, 128)])
        return c

    lax.fori_loop(0, cntp // 128, final_flush, 0)
    cntv[...] = lax.broadcast(off + cntp, (16,))
    pltpu.sync_copy(cntv, cnt_hbm.at[wid])


def _sc_filter(src, dst):
    mesh = plsc.VectorSubcoreMesh(core_axis_name="c", subcore_axis_name="s")
    kfn = pl.kernel(
        _filter_body,
        out_type=[jax.ShapeDtypeStruct((NW, E_EDGES), jnp.int32),
                  jax.ShapeDtypeStruct((NW, E_EDGES), jnp.int32),
                  jax.ShapeDtypeStruct((NW, 16), jnp.int32)],
        mesh=mesh,
        scratch_types=[
            pltpu.VMEM((2, SG), jnp.int32),
            pltpu.VMEM((2, SG), jnp.int32),
            pltpu.VMEM((FB,), jnp.int32),
            pltpu.VMEM((FB,), jnp.int32),
            pltpu.VMEM((16,), jnp.int32),
            pltpu.SemaphoreType.DMA,
            pltpu.SemaphoreType.DMA,
        ],
        compiler_params=pltpu.CompilerParams(use_tc_tiling_on_sc=False,
                                             needs_layout_passes=False),
    )
    return kfn(src, dst)


def _segmax_body(v_hbm, srcl_hbm, dlocl_hbm, cnt_hbm, m_hbm,
                 idx_v, rows_v, dstv_v, acc_v, stv_v,
                 sem_g0, sem_g1, sem_i0, sem_i1, sem_d0, sem_d1):
    wid = lax.axis_index("s") * 2 + lax.axis_index("c")
    lo = wid * RPW
    pltpu.sync_copy(cnt_hbm.at[wid], stv_v)

    neg = jnp.full((16,), NEG, jnp.float32)

    @pl.loop(0, RPW + 16)
    def _(r):
        for f in range(4):
            acc_v[r, pl.ds(f * 16, 16)] = neg

    cnt = stv_v[pl.ds(0, 16)]
    c0 = 0
    c1 = cnt[0] // G

    sem_g = (sem_g0, sem_g1)
    sem_i = (sem_i0, sem_i1)
    sem_d = (sem_d0, sem_d1)

    def idx_dma(ci, b):
        co = pl.multiple_of(ci * G, G)
        return (pltpu.make_async_copy(srcl_hbm.at[wid, pl.ds(co, G)],
                                      idx_v.at[b], sem_i[b]),
                pltpu.make_async_copy(dlocl_hbm.at[wid, pl.ds(co, G)],
                                      dstv_v.at[b], sem_d[b]))

    def gather(b):
        return pltpu.make_async_copy(v_hbm.at[idx_v.at[b]], rows_v.at[b],
                                     sem_g[b])

    def process(ci, b):
        def grp(gi, carry2):
            d16 = dstv_v[b, pl.ds(gi * 16, 16)]
            for lane in range(16):
                dl = d16[lane]
                ei = gi * 16 + lane
                for f in range(4):
                    sl = pl.ds(f * 16, 16)
                    acc_v[dl, sl] = jnp.maximum(acc_v[dl, sl],
                                                rows_v[b, ei, sl])

            return carry2

        lax.fori_loop(0, G // 16, grp, 0)

    @pl.when(c0 < c1)
    def _():
        ia, da = idx_dma(c0, 0)
        ia.start()
        da.start()
        ia.wait()
        gather(0).start()

        @pl.when(c0 + 1 < c1)
        def _():
            ib, db = idx_dma(c0 + 1, 1)
            ib.start()
            db.start()

        def pair(t, carry):
            for b in range(2):
                ci = c0 + 2 * t + b

                @pl.when(ci < c1)
                def _():
                    gather(b).wait()
                    _, dw = idx_dma(ci, b)
                    dw.wait()

                    @pl.when(ci + 1 < c1)
                    def _():
                        iw, _ = idx_dma(ci + 1, 1 - b)
                        iw.wait()
                        gather(1 - b).start()

                    @pl.when(ci + 2 < c1)
                    def _():
                        inx, _ = idx_dma(ci + 2, b)
                        inx.start()

                    process(ci, b)

                    @pl.when(ci + 2 < c1)
                    def _():
                        _, dnx = idx_dma(ci + 2, b)
                        dnx.start()

            return carry

        npairs = lax.div(c1 - c0 + 1, 2)
        lax.fori_loop(0, npairs, pair, 0)

    pltpu.sync_copy(acc_v.at[pl.ds(0, RPW)], m_hbm.at[pl.ds(lo, RPW)])


def _sc_segmax(v, srcl, dlocl, cnt):
    mesh = plsc.VectorSubcoreMesh(core_axis_name="c", subcore_axis_name="s")
    kfn = pl.kernel(
        _segmax_body,
        out_type=jax.ShapeDtypeStruct((NP, 64), jnp.float32),
        mesh=mesh,
        scratch_types=[
            pltpu.VMEM((2, G), jnp.int32),
            pltpu.VMEM((2, G, 64), jnp.float32),
            pltpu.VMEM((2, G), jnp.int32),
            pltpu.VMEM((RPW + 16, 64), jnp.float32),
            pltpu.VMEM((16,), jnp.int32),
            pltpu.SemaphoreType.DMA,
            pltpu.SemaphoreType.DMA,
            pltpu.SemaphoreType.DMA,
            pltpu.SemaphoreType.DMA,
            pltpu.SemaphoreType.DMA,
            pltpu.SemaphoreType.DMA,
        ],
        compiler_params=pltpu.CompilerParams(use_tc_tiling_on_sc=False),
    )
    return kfn(v, srcl, dlocl, cnt)


def kernel(x, edge_index, W1, b1, W2, b2, W3, b3, W4, b4, W5, b5,
           Wl1, bl1, Wl2, bl2):
    f32 = jnp.float32
    src = edge_index[0].astype(jnp.int32)
    dst = edge_index[1].astype(jnp.int32)

    srcl, dlocl, cnt = _sc_filter(src, dst)

    xpad = jnp.zeros((NP, 8), f32).at[:N_NODES, :3].set(x)

    wd1 = jnp.zeros((8, 64), f32).at[:3].set(W1[:3] - W1[3:])
    wb1 = jnp.zeros((8, 64), f32).at[:3].set(W1[3:])
    u, v = _tc_uv(xpad, wd1, wb1)
    m = _sc_segmax(v, srcl, dlocl, cnt)
    bprev = b1

    for (W, b) in ((W2, b2), (W3, b3), (W4, b4), (W5, b5)):
        wd = W[:64] - W[64:]
        wb = W[64:]
        b2d = jnp.broadcast_to(bprev.reshape(1, 64), (8, 64))
        u, v = _tc_mid(u, m, b2d, wd, wb)
        m = _sc_segmax(v, srcl, dlocl, cnt)
        bprev = b

    b2d5 = jnp.broadcast_to(bprev.reshape(1, 64), (8, 64))
    x5, g8 = _tc_x5g(u, m, b2d5)

    amat = Wl1[:64]
    bmat = Wl1[64:]
    bl1_2d = jnp.broadcast_to(bl1.reshape(1, 128), (8, 128))
    wl2p = jnp.zeros((128, 8), f32).at[:, :3].set(Wl2)
    bl2_2d = jnp.zeros((8, 8), f32).at[:, :3].set(
        jnp.broadcast_to(bl2.reshape(1, 3), (8, 3)))
    outp = _tc_final(x5, g8, xpad, amat, bmat, bl1_2d, wl2p, bl2_2d)
    return outp[:N_NODES, :3]
